# baseline (speedup 1.0000x reference)
import functools

import jax
import jax.numpy as jnp
from jax import lax
from jax.experimental import pallas as pl
from jax.experimental.pallas import tpu as pltpu
from jax.experimental.pallas import tpu_sc as plsc

N_SITES = 100000
K = 64
B = 16384
NC, NS, L = 2, 16, 16
NW = NC * NS
BPW = B // NW
CHUNK = 128
NCH = BPW // CHUNK

_D0 = 0.0009223163497825149
_D1 = 0.9797691943591391
_D2 = -0.3935581873890316
_D3 = 0.10669243657177084


def _softplus16(x):
    t = jnp.exp(-jnp.abs(x))
    p = _D2 + t * _D3
    p = _D1 + t * p
    p = _D0 + t * p
    return jnp.maximum(x, 0.0) + p


@functools.cache
def _get_mesh():
    return plsc.VectorSubcoreMesh(
        core_axis_name="c", subcore_axis_name="s", num_cores=NC, num_subcores=NS
    )


def _hbb_body(idx_hbm, qab_hbm, out_hbm, idx_v, rows_v, gsems, wsem):
    wid = lax.axis_index("s") * NC + lax.axis_index("c")

    pltpu.sync_copy(idx_hbm.at[wid], idx_v)

    gathers = []
    for j in range(NCH):
        gathers.append(
            pltpu.async_copy(
                qab_hbm.at[idx_v.at[j]],
                rows_v.at[pl.ds(j * CHUNK, CHUNK)],
                gsems.at[j],
            )
        )

    writes = []
    for j in range(NCH):
        gathers[j].wait()

        @plsc.parallel_loop(j * CHUNK, (j + 1) * CHUNK, unroll=2)
        def _(r):
            for c in range(K // L):
                a = _softplus16(rows_v[r, pl.ds(c * L, L)])
                b = _softplus16(rows_v[r, pl.ds(K + c * L, L)])
                rows_v[r, pl.ds(c * L, L)] = a / (a + b)

        sl = pl.ds(j * CHUNK, CHUNK)
        writes.append(
            pltpu.async_copy(
                rows_v.at[sl], out_hbm.at[pl.ds(wid * BPW + j * CHUNK, CHUNK)], wsem
            )
        )

    for wr in writes:
        wr.wait()


@functools.cache
def _get_hbb_sc():
    return functools.partial(
        pl.kernel,
        out_type=jax.ShapeDtypeStruct((B, 2 * K), jnp.float32),
        mesh=_get_mesh(),
        scratch_types=[
            pltpu.VMEM((NCH, CHUNK), jnp.int32),
            pltpu.VMEM((BPW, 2 * K), jnp.float32),
            pltpu.SemaphoreType.DMA((NCH,)),
            pltpu.SemaphoreType.DMA,
        ],
    )(_hbb_body)


_BLKC = 2000


def _concat_body(qa_ref, qb_ref, out_ref):
    out_ref[...] = jnp.concatenate((qa_ref[...], qb_ref[...]), axis=-1)


@functools.cache
def _get_concat_tc():
    return pl.pallas_call(
        _concat_body,
        grid=(N_SITES // _BLKC,),
        in_specs=[
            pl.BlockSpec((_BLKC, K), lambda i: (i, 0)),
            pl.BlockSpec((_BLKC, K), lambda i: (i, 0)),
        ],
        out_specs=pl.BlockSpec((_BLKC, 2 * K), lambda i: (i, 0)),
        out_shape=jax.ShapeDtypeStruct((N_SITES, 2 * K), jnp.float32),
    )


_BLKS = 2048


def _slice_body(wide_ref, out_ref):
    out_ref[...] = wide_ref[:, :K]


@functools.cache
def _get_slice_tc():
    return pl.pallas_call(
        _slice_body,
        grid=(B // _BLKS,),
        in_specs=[pl.BlockSpec((_BLKS, 2 * K), lambda i: (i, 0))],
        out_specs=pl.BlockSpec((_BLKS, K), lambda i: (i, 0)),
        out_shape=jax.ShapeDtypeStruct((B, K), jnp.float32),
    )


def kernel(site_idx, q_a_site, q_b_site):
    qab = _get_concat_tc()(q_a_site, q_b_site)
    idx = site_idx.astype(jnp.int32).reshape(NW, NCH, CHUNK)
    wide = _get_hbb_sc()(idx, qab)
    return _get_slice_tc()(wide)
